# baseline (speedup 1.0000x reference)
import jax
import jax.numpy as jnp
from jax.experimental import pallas as pl
from jax.experimental.pallas import tpu as pltpu

_M = 200


def _proj_body(feat_ref, wt_ref, p1_ref, p2_ref):
    f = feat_ref[...]
    wt = wt_ref[...]
    d_in = f.shape[1]
    p1_ref[...] = jnp.dot(f, wt[:d_in, :], preferred_element_type=jnp.float32)
    p2_ref[...] = jnp.dot(f, wt[d_in:, :], preferred_element_type=jnp.float32)


def _main_body(p1_ref, adj_ref, p2_ref, out_ref):
    a = adj_ref[...]
    acc = jnp.dot(a, p2_ref[...], preferred_element_type=jnp.float32)
    denom = jnp.sum(a, axis=1, keepdims=True) + 1.0
    out_ref[...] = jnp.maximum(p1_ref[...] + acc / denom, 0.0)


def kernel(features, adj, W):
    n, d_in = features.shape
    d_out = W.shape[0]
    wt = W.T

    p1, p2 = pl.pallas_call(
        _proj_body,
        out_shape=(
            jax.ShapeDtypeStruct((n, d_out), jnp.float32),
            jax.ShapeDtypeStruct((n, d_out), jnp.float32),
        ),
    )(features, wt)

    out = pl.pallas_call(
        _main_body,
        grid=(n // _M,),
        in_specs=[
            pl.BlockSpec((_M, d_out), lambda i: (i, 0)),
            pl.BlockSpec((_M, n), lambda i: (i, 0)),
            pl.BlockSpec((n, d_out), lambda i: (0, 0)),
        ],
        out_specs=pl.BlockSpec((_M, d_out), lambda i: (i, 0)),
        out_shape=jax.ShapeDtypeStruct((n, d_out), jnp.float32),
        compiler_params=pltpu.CompilerParams(
            dimension_semantics=("parallel",),
        ),
    )(p1, adj, p2)
    return out


# speedup vs baseline: 1.0783x; 1.0783x over previous
import jax
import jax.numpy as jnp
from jax.experimental import pallas as pl
from jax.experimental.pallas import tpu as pltpu

_M = 400


def _main_body(adj_ref, feat_ref, wt_ref, out_ref, p2_ref):
    i = pl.program_id(0)
    d_in = feat_ref.shape[1]

    @pl.when(i == 0)
    def _():
        p2_ref[...] = jnp.dot(
            feat_ref[...], wt_ref[d_in:, :], preferred_element_type=jnp.float32
        )

    a = adj_ref[...]
    acc = jnp.dot(a, p2_ref[...], preferred_element_type=jnp.float32)
    denom = jnp.sum(a, axis=1, keepdims=True) + 1.0
    f_rows = feat_ref[pl.ds(i * _M, _M), :]
    p1 = jnp.dot(f_rows, wt_ref[:d_in, :], preferred_element_type=jnp.float32)
    out_ref[...] = jnp.maximum(p1 + acc / denom, 0.0)


def kernel(features, adj, W):
    n, d_in = features.shape
    d_out = W.shape[0]
    wt = W.T

    out = pl.pallas_call(
        _main_body,
        grid=(n // _M,),
        in_specs=[
            pl.BlockSpec((_M, n), lambda i: (i, 0)),
            pl.BlockSpec((n, d_in), lambda i: (0, 0)),
            pl.BlockSpec((2 * d_in, d_out), lambda i: (0, 0)),
        ],
        out_specs=pl.BlockSpec((_M, d_out), lambda i: (i, 0)),
        out_shape=jax.ShapeDtypeStruct((n, d_out), jnp.float32),
        scratch_shapes=[pltpu.VMEM((n, d_out), jnp.float32)],
        compiler_params=pltpu.CompilerParams(
            dimension_semantics=("arbitrary",),
        ),
    )(adj, features, wt)
    return out
